# baseline (speedup 1.0000x reference)
import jax
import jax.numpy as jnp
from jax.experimental import pallas as pl


def _gcn(x, edge_index, W, b, num_nodes):
    src = edge_index[0]
    dst = edge_index[1]
    loop = jnp.arange(num_nodes, dtype=src.dtype)
    src = jnp.concatenate([src, loop])
    dst = jnp.concatenate([dst, loop])
    xw = x @ W
    deg = jnp.zeros((num_nodes,), dtype=x.dtype).at[dst].add(1.0)
    dinv = jax.lax.rsqrt(jnp.maximum(deg, 1.0))
    norm = dinv[src] * dinv[dst]
    msg = jnp.take(xw, src, axis=0) * norm[:, None]
    out = jax.ops.segment_sum(msg, dst, num_segments=num_nodes)
    return out + b


def _mlp_body(px_ref, dx_ref, w1_ref, b1_ref, w2_ref, b2_ref, o_ref):
    z = px_ref[...] @ w1_ref[0] + dx_ref[...] @ w1_ref[1] + b1_ref[...]
    h = jnp.maximum(z, 0.0)
    o_ref[...] = h @ w2_ref[...] + b2_ref[...]


def kernel(protein_x, protein_edge_index, drug_x, drug_edge_index,
           Wp1, bp1, Wp2, bp2, Wd1, bd1, Wd2, bd2, Wfc1, bfc1, Wfc2, bfc2):
    N = 10000
    px = jax.nn.relu(_gcn(protein_x, protein_edge_index, Wp1, bp1, N))
    px = jax.nn.relu(_gcn(px, protein_edge_index, Wp2, bp2, N))
    px = px.mean(axis=0, keepdims=True)
    dx = jax.nn.relu(_gcn(drug_x, drug_edge_index, Wd1, bd1, N))
    dx = jax.nn.relu(_gcn(dx, drug_edge_index, Wd2, bd2, N))
    dx = dx.mean(axis=0, keepdims=True)
    w1 = Wfc1.reshape(2, 128, 128)
    out = pl.pallas_call(
        _mlp_body,
        out_shape=jax.ShapeDtypeStruct((1, 1), jnp.float32),
    )(px, dx, w1, bfc1.reshape(1, 128), Wfc2, bfc2.reshape(1, 1))
    return out.reshape(1)


# speedup vs baseline: 5.8280x; 5.8280x over previous
import functools

import jax
import jax.numpy as jnp
from jax import lax
from jax.experimental import pallas as pl
from jax.experimental.pallas import tpu as pltpu
from jax.experimental.pallas import tpu_sc as plsc

N = 10000
D = 128
NP = 10240
ACC = 12288
NB = NP // 128
NW = 32


def _fill(ref, rows, cols, value):
    v = jnp.full((16,), value, jnp.float32)

    def body(i, c):
        r = i // (cols // 16)
        j = i % (cols // 16)
        ref[r, pl.ds(j * 16, 16)] = v
        return c

    lax.fori_loop(0, rows * (cols // 16), body, 0)


def _seg_body(y, src2d, dst2d, out, sidx, didx, rows, acc, sem):
    cid = lax.axis_index("c")
    sid = lax.axis_index("s")
    wid = sid * 2 + cid
    _fill(rows, 128, 128, 0.0)

    @pl.loop(0, ACC // 16 // 128)
    def _(j):
        pltpu.async_copy(
            rows, acc.at[pl.ds(sid * (ACC // 16) + j * 128, 128)],
            sem).wait()

    plsc.subcore_barrier()

    nchunk = src2d.shape[0] // NW

    @pl.loop(0, nchunk)
    def _(g):
        r = wid * nchunk + g
        pltpu.async_copy(src2d.at[r], sidx, sem).wait()
        pltpu.async_copy(y.at[sidx], rows, sem).wait()
        pltpu.async_copy(dst2d.at[r], didx, sem).wait()
        pltpu.async_copy(rows, acc.at[didx], sem, add=True).wait()

    plsc.subcore_barrier()

    @pl.loop(0, NP // 16 // 128)
    def _(j):
        r0 = sid * (NP // 16) + j * 128
        pltpu.async_copy(acc.at[pl.ds(r0, 128)], rows, sem).wait()
        pltpu.async_copy(rows, out.at[cid, pl.ds(r0, 128)], sem).wait()


def _seg_call(y, src2d, dst2d):
    k = functools.partial(
        pl.kernel,
        out_type=jax.ShapeDtypeStruct((2, NP, D), jnp.float32),
        mesh=plsc.VectorSubcoreMesh(core_axis_name="c", subcore_axis_name="s"),
        scratch_types=[
            pltpu.VMEM((128,), jnp.int32),
            pltpu.VMEM((128,), jnp.int32),
            pltpu.VMEM((128, D), jnp.float32),
            pltpu.VMEM_SHARED((ACC, D), jnp.float32),
            pltpu.SemaphoreType.DMA,
        ],
    )(_seg_body)
    return k(y, src2d, dst2d)


def _dinv_body(s_ref, o_ref):
    deg = s_ref[0, :, 0:1] + s_ref[1, :, 0:1] + 1.0
    o_ref[...] = lax.rsqrt(deg)


def _dinv_call(S):
    return pl.pallas_call(
        _dinv_body,
        grid=(NB,),
        in_specs=[pl.BlockSpec((2, 128, D), lambda g: (0, g, 0))],
        out_specs=pl.BlockSpec((128, 1), lambda g: (g, 0)),
        out_shape=jax.ShapeDtypeStruct((NP, 1), jnp.float32),
    )(S)


def _k1_body(x_ref, w_ref, dinv_ref, o_ref):
    o_ref[...] = jnp.dot(x_ref[...], w_ref[...],
                         preferred_element_type=jnp.float32) * dinv_ref[...]


def _k1_call(xpad, W, dinv):
    return pl.pallas_call(
        _k1_body,
        grid=(NB,),
        in_specs=[
            pl.BlockSpec((128, D), lambda g: (g, 0)),
            pl.BlockSpec((D, D), lambda g: (0, 0)),
            pl.BlockSpec((128, 1), lambda g: (g, 0)),
        ],
        out_specs=pl.BlockSpec((128, D), lambda g: (g, 0)),
        out_shape=jax.ShapeDtypeStruct((NP, D), jnp.float32),
    )(xpad, W, dinv)


def _k2_body(y_ref, s_ref, dinv_ref, b_ref, w_ref, o_ref):
    s = s_ref[0] + s_ref[1]
    dinv = dinv_ref[...]
    h = jnp.maximum(dinv * (y_ref[...] + s) + b_ref[...], 0.0)
    o_ref[...] = jnp.dot(h, w_ref[...],
                         preferred_element_type=jnp.float32) * dinv


def _k2_call(y, S, dinv, b, W2):
    return pl.pallas_call(
        _k2_body,
        grid=(NB,),
        in_specs=[
            pl.BlockSpec((128, D), lambda g: (g, 0)),
            pl.BlockSpec((2, 128, D), lambda g: (0, g, 0)),
            pl.BlockSpec((128, 1), lambda g: (g, 0)),
            pl.BlockSpec((1, D), lambda g: (0, 0)),
            pl.BlockSpec((D, D), lambda g: (0, 0)),
        ],
        out_specs=pl.BlockSpec((128, D), lambda g: (g, 0)),
        out_shape=jax.ShapeDtypeStruct((NP, D), jnp.float32),
    )(y, S, dinv, b.reshape(1, D), W2)


def _k3_body(y_ref, s_ref, dinv_ref, b_ref, o_ref):
    g = pl.program_id(0)
    s = s_ref[0] + s_ref[1]
    h = jnp.maximum(dinv_ref[...] * (y_ref[...] + s) + b_ref[...], 0.0)
    r = g * 128 + lax.broadcasted_iota(jnp.int32, (128, 1), 0)
    h = jnp.where(r < N, h, 0.0)
    part = jnp.sum(h, axis=0, keepdims=True)

    @pl.when(g == 0)
    def _():
        o_ref[...] = part

    @pl.when(g > 0)
    def _():
        o_ref[...] = o_ref[...] + part


def _k3_call(y2, S2, dinv, b):
    return pl.pallas_call(
        _k3_body,
        grid=(NB,),
        in_specs=[
            pl.BlockSpec((128, D), lambda g: (g, 0)),
            pl.BlockSpec((2, 128, D), lambda g: (0, g, 0)),
            pl.BlockSpec((128, 1), lambda g: (g, 0)),
            pl.BlockSpec((1, D), lambda g: (0, 0)),
        ],
        out_specs=pl.BlockSpec((1, D), lambda g: (0, 0)),
        out_shape=jax.ShapeDtypeStruct((1, D), jnp.float32),
    )(y2, S2, dinv, b.reshape(1, D))


def _k4_body(pp_ref, pd_ref, w1_ref, b1_ref, w2_ref, b2_ref, o_ref):
    inv_n = jnp.float32(1.0 / N)
    z = (jnp.dot(pp_ref[...] * inv_n, w1_ref[0],
                 preferred_element_type=jnp.float32)
         + jnp.dot(pd_ref[...] * inv_n, w1_ref[1],
                   preferred_element_type=jnp.float32)
         + b1_ref[...])
    h = jnp.maximum(z, 0.0)
    o_ref[...] = jnp.dot(h, w2_ref[...],
                         preferred_element_type=jnp.float32) + b2_ref[...]


def _pad_edges(e, mult, trash):
    src = e[0].astype(jnp.int32)
    dst = e[1].astype(jnp.int32)
    ne = src.shape[0]
    npad = -ne % mult
    src = jnp.concatenate([src, jnp.zeros((npad,), jnp.int32)])
    dst = jnp.concatenate([dst, jnp.full((npad,), trash, jnp.int32)])
    return src.reshape(-1, 128), dst.reshape(-1, 128)


def kernel(protein_x, protein_edge_index, drug_x, drug_edge_index,
           Wp1, bp1, Wp2, bp2, Wd1, bd1, Wd2, bd2, Wfc1, bfc1, Wfc2, bfc2):
    psrc, pdst = _pad_edges(protein_edge_index, NW * 128, NP)
    dsrc, ddst = _pad_edges(drug_edge_index, NW * 128, NP)
    xp = jnp.pad(protein_x, ((0, NP - N), (0, 0)))
    xd = jnp.pad(drug_x, ((0, NP - N), (0, 0)))
    ones = jnp.ones((NP, D), jnp.float32)

    dinv_p = _dinv_call(_seg_call(ones, psrc, pdst))
    dinv_d = _dinv_call(_seg_call(ones, dsrc, ddst))

    y1p = _k1_call(xp, Wp1, dinv_p)
    S1p = _seg_call(y1p, psrc, pdst)
    y2p = _k2_call(y1p, S1p, dinv_p, bp1, Wp2)
    S2p = _seg_call(y2p, psrc, pdst)
    pp = _k3_call(y2p, S2p, dinv_p, bp2)

    y1d = _k1_call(xd, Wd1, dinv_d)
    S1d = _seg_call(y1d, dsrc, ddst)
    y2d = _k2_call(y1d, S1d, dinv_d, bd1, Wd2)
    S2d = _seg_call(y2d, dsrc, ddst)
    pd = _k3_call(y2d, S2d, dinv_d, bd2)

    out = pl.pallas_call(
        _k4_body,
        out_shape=jax.ShapeDtypeStruct((1, 1), jnp.float32),
    )(pp, pd, Wfc1.reshape(2, D, D), bfc1.reshape(1, D), Wfc2,
      bfc2.reshape(1, 1))
    return out.reshape(1)


# speedup vs baseline: 7.4269x; 1.2743x over previous
import functools

import jax
import jax.numpy as jnp
from jax import lax
from jax.experimental import pallas as pl
from jax.experimental.pallas import tpu as pltpu
from jax.experimental.pallas import tpu_sc as plsc

N = 10000
D = 128
NP = 10240
ACC = 10240
NB = NP // 128
NW = 32


def _fill(ref, rows, cols, value):
    v = jnp.full((16,), value, jnp.float32)

    def body(i, c):
        r = i // (cols // 16)
        j = i % (cols // 16)
        ref[r, pl.ds(j * 16, 16)] = v
        return c

    lax.fori_loop(0, rows * (cols // 16), body, 0)


def _seg_body(y, src2d, dst2d, out, sidxb, didx0, didx1, rows0, rows1, acc,
              gs0, gs1, ds0, ds1, ss):
    cid = lax.axis_index("c")
    sid = lax.axis_index("s")
    wid = sid * 2 + cid
    _fill(rows0, 128, 128, 0.0)

    @pl.loop(0, ACC // 16 // 128)
    def _(j):
        pltpu.async_copy(
            rows0, acc.at[pl.ds(sid * (ACC // 16) + j * 128, 128)],
            ss).wait()

    plsc.subcore_barrier()

    nchunk = src2d.shape[0] // 128 // NW
    hc = sidxb.shape[0] // 128
    base = wid * nchunk

    def gather(b, j, rbuf, gsem):
        pltpu.async_copy(y.at[sidxb.at[pl.ds(j * 128, 128)]], rbuf, gsem)
        pltpu.async_copy(dst2d.at[b + j], didx0 if rbuf is rows0 else didx1,
                         ds0 if rbuf is rows0 else ds1)

    def scatter(b, rbuf, gsem, dsem, dbuf):
        pltpu.make_async_copy(y.at[sidxb.at[pl.ds(0, 128)]], rbuf,
                              gsem).wait()
        pltpu.make_async_copy(dst2d.at[b], dbuf, dsem).wait()
        pltpu.async_copy(rbuf, acc.at[dbuf], ss, add=True).wait()

    @pl.loop(0, nchunk // hc)
    def _(h):
        hb = base + h * hc
        pltpu.async_copy(src2d.at[pl.ds(hb * 128, hc * 128)], sidxb,
                         ss).wait()
        gather(hb, 0, rows0, gs0)

        @pl.loop(0, hc // 2 - 1)
        def _(t):
            gather(hb, 2 * t + 1, rows1, gs1)
            scatter(hb, rows0, gs0, ds0, didx0)
            gather(hb, 2 * t + 2, rows0, gs0)
            scatter(hb, rows1, gs1, ds1, didx1)

        gather(hb, hc - 1, rows1, gs1)
        scatter(hb, rows0, gs0, ds0, didx0)
        scatter(hb, rows1, gs1, ds1, didx1)

    plsc.subcore_barrier()

    @pl.loop(0, ACC // 16 // 128)
    def _(j):
        r0 = sid * (ACC // 16) + j * 128
        pltpu.async_copy(acc.at[pl.ds(r0, 128)], rows0, ss).wait()
        pltpu.async_copy(rows0, out.at[cid, pl.ds(r0, 128)], ss).wait()


def _seg_call(y, src2d, dst2d):
    hc = src2d.shape[0] // NW // 2
    k = functools.partial(
        pl.kernel,
        out_type=jax.ShapeDtypeStruct((2, NP, D), jnp.float32),
        mesh=plsc.VectorSubcoreMesh(core_axis_name="c", subcore_axis_name="s"),
        scratch_types=[
            pltpu.VMEM((hc * 128,), jnp.int32),
            pltpu.VMEM((128,), jnp.int32),
            pltpu.VMEM((128,), jnp.int32),
            pltpu.VMEM((128, D), jnp.float32),
            pltpu.VMEM((128, D), jnp.float32),
            pltpu.VMEM_SHARED((ACC, D), jnp.float32),
            pltpu.SemaphoreType.DMA,
            pltpu.SemaphoreType.DMA,
            pltpu.SemaphoreType.DMA,
            pltpu.SemaphoreType.DMA,
            pltpu.SemaphoreType.DMA,
        ],
    )(_seg_body)
    return k(y, src2d.reshape(-1), dst2d)


def _dinv_body(s_ref, o_ref):
    deg = s_ref[0, :, 0:1] + s_ref[1, :, 0:1] + 1.0
    o_ref[...] = lax.rsqrt(deg)


def _dinv_call(S):
    return pl.pallas_call(
        _dinv_body,
        grid=(NB,),
        in_specs=[pl.BlockSpec((2, 128, D), lambda g: (0, g, 0))],
        out_specs=pl.BlockSpec((128, 1), lambda g: (g, 0)),
        out_shape=jax.ShapeDtypeStruct((NP, 1), jnp.float32),
    )(S)


def _k1_body(x_ref, w_ref, dinv_ref, o_ref):
    o_ref[...] = jnp.dot(x_ref[...], w_ref[...],
                         preferred_element_type=jnp.float32) * dinv_ref[...]


def _k1_call(xpad, W, dinv):
    return pl.pallas_call(
        _k1_body,
        grid=(NB,),
        in_specs=[
            pl.BlockSpec((128, D), lambda g: (g, 0)),
            pl.BlockSpec((D, D), lambda g: (0, 0)),
            pl.BlockSpec((128, 1), lambda g: (g, 0)),
        ],
        out_specs=pl.BlockSpec((128, D), lambda g: (g, 0)),
        out_shape=jax.ShapeDtypeStruct((NP, D), jnp.float32),
    )(xpad, W, dinv)


def _k2_body(y_ref, s_ref, dinv_ref, b_ref, w_ref, o_ref):
    s = s_ref[0] + s_ref[1]
    dinv = dinv_ref[...]
    h = jnp.maximum(dinv * (y_ref[...] + s) + b_ref[...], 0.0)
    o_ref[...] = jnp.dot(h, w_ref[...],
                         preferred_element_type=jnp.float32) * dinv


def _k2_call(y, S, dinv, b, W2):
    return pl.pallas_call(
        _k2_body,
        grid=(NB,),
        in_specs=[
            pl.BlockSpec((128, D), lambda g: (g, 0)),
            pl.BlockSpec((2, 128, D), lambda g: (0, g, 0)),
            pl.BlockSpec((128, 1), lambda g: (g, 0)),
            pl.BlockSpec((1, D), lambda g: (0, 0)),
            pl.BlockSpec((D, D), lambda g: (0, 0)),
        ],
        out_specs=pl.BlockSpec((128, D), lambda g: (g, 0)),
        out_shape=jax.ShapeDtypeStruct((NP, D), jnp.float32),
    )(y, S, dinv, b.reshape(1, D), W2)


def _k3_body(y_ref, s_ref, dinv_ref, b_ref, o_ref):
    g = pl.program_id(0)
    s = s_ref[0] + s_ref[1]
    h = jnp.maximum(dinv_ref[...] * (y_ref[...] + s) + b_ref[...], 0.0)
    r = g * 128 + lax.broadcasted_iota(jnp.int32, (128, 1), 0)
    h = jnp.where(r < N, h, 0.0)
    part = jnp.sum(h, axis=0, keepdims=True)

    @pl.when(g == 0)
    def _():
        o_ref[...] = part

    @pl.when(g > 0)
    def _():
        o_ref[...] = o_ref[...] + part


def _k3_call(y2, S2, dinv, b):
    return pl.pallas_call(
        _k3_body,
        grid=(NB,),
        in_specs=[
            pl.BlockSpec((128, D), lambda g: (g, 0)),
            pl.BlockSpec((2, 128, D), lambda g: (0, g, 0)),
            pl.BlockSpec((128, 1), lambda g: (g, 0)),
            pl.BlockSpec((1, D), lambda g: (0, 0)),
        ],
        out_specs=pl.BlockSpec((1, D), lambda g: (0, 0)),
        out_shape=jax.ShapeDtypeStruct((1, D), jnp.float32),
    )(y2, S2, dinv, b.reshape(1, D))


def _k4_body(pp_ref, pd_ref, w1_ref, b1_ref, w2_ref, b2_ref, o_ref):
    inv_n = jnp.float32(1.0 / N)
    z = (jnp.dot(pp_ref[...] * inv_n, w1_ref[0],
                 preferred_element_type=jnp.float32)
         + jnp.dot(pd_ref[...] * inv_n, w1_ref[1],
                   preferred_element_type=jnp.float32)
         + b1_ref[...])
    h = jnp.maximum(z, 0.0)
    o_ref[...] = jnp.dot(h, w2_ref[...],
                         preferred_element_type=jnp.float32) + b2_ref[...]


def _pad_edges(e, mult, trash):
    src = e[0].astype(jnp.int32)
    dst = e[1].astype(jnp.int32)
    ne = src.shape[0]
    npad = -ne % mult
    src = jnp.concatenate([src, jnp.zeros((npad,), jnp.int32)])
    dst = jnp.concatenate([dst, jnp.full((npad,), trash, jnp.int32)])
    return src.reshape(-1, 128), dst.reshape(-1, 128)


def kernel(protein_x, protein_edge_index, drug_x, drug_edge_index,
           Wp1, bp1, Wp2, bp2, Wd1, bd1, Wd2, bd2, Wfc1, bfc1, Wfc2, bfc2):
    psrc, pdst = _pad_edges(protein_edge_index, NW * 128, N + 16)
    dsrc, ddst = _pad_edges(drug_edge_index, NW * 128, N + 16)
    xp = jnp.pad(protein_x, ((0, NP - N), (0, 0)))
    xd = jnp.pad(drug_x, ((0, NP - N), (0, 0)))
    ones = jnp.ones((NP, D), jnp.float32)

    dinv_p = _dinv_call(_seg_call(ones, psrc, pdst))
    dinv_d = _dinv_call(_seg_call(ones, dsrc, ddst))

    y1p = _k1_call(xp, Wp1, dinv_p)
    S1p = _seg_call(y1p, psrc, pdst)
    y2p = _k2_call(y1p, S1p, dinv_p, bp1, Wp2)
    S2p = _seg_call(y2p, psrc, pdst)
    pp = _k3_call(y2p, S2p, dinv_p, bp2)

    y1d = _k1_call(xd, Wd1, dinv_d)
    S1d = _seg_call(y1d, dsrc, ddst)
    y2d = _k2_call(y1d, S1d, dinv_d, bd1, Wd2)
    S2d = _seg_call(y2d, dsrc, ddst)
    pd = _k3_call(y2d, S2d, dinv_d, bd2)

    out = pl.pallas_call(
        _k4_body,
        out_shape=jax.ShapeDtypeStruct((1, 1), jnp.float32),
    )(pp, pd, Wfc1.reshape(2, D, D), bfc1.reshape(1, D), Wfc2,
      bfc2.reshape(1, 1))
    return out.reshape(1)
